# Initial kernel scaffold; baseline (speedup 1.0000x reference)
import jax
import jax.numpy as jnp
from jax.experimental import pallas as pl


def kernel(hidden_states, W_router, b_router, gate_up_proj, gate_up_proj_bias, down_proj, down_proj_bias):
    raise NotImplementedError("write your pallas kernel here")


# speedup vs baseline: 23.2924x; 23.2924x over previous
import functools

import jax
import jax.numpy as jnp
from jax.experimental import pallas as pl
from jax.experimental.pallas import tpu as pltpu

H = 1024
FF = 512
E = 8
ALPHA = 1.702
LIMIT = 7.0


def _moe_body(x_ref, wr_ref, br_ref, w1_ref, b1_ref, w2_ref, b2_ref,
              o_ref, wdense_ref):
    e = pl.program_id(0)
    x = x_ref[...]
    T = x.shape[0]

    @pl.when(e == 0)
    def _():
        logits = jnp.dot(x, wr_ref[...], preferred_element_type=jnp.float32)
        logits = logits + br_ref[...]
        m = jnp.max(logits, axis=1, keepdims=True)
        ex = jnp.exp(logits - m)
        probs = ex / jnp.sum(ex, axis=1, keepdims=True)
        eidx = jax.lax.broadcasted_iota(jnp.int32, probs.shape, 1)
        m1 = jnp.max(probs, axis=1, keepdims=True)
        a1 = jnp.min(jnp.where(probs >= m1, eidx, E), axis=1, keepdims=True)
        mask1 = eidx == a1
        probsb = jnp.where(mask1, -jnp.inf, probs)
        m2 = jnp.max(probsb, axis=1, keepdims=True)
        a2 = jnp.min(jnp.where(probsb >= m2, eidx, E), axis=1, keepdims=True)
        mask2 = eidx == a2
        s = m1 + m2 + 1e-20
        wdense_ref[...] = (jnp.where(mask1, m1 / s, 0.0)
                           + jnp.where(mask2, m2 / s, 0.0))
        o_ref[...] = jnp.zeros_like(o_ref)

    v = jnp.dot(x, w1_ref[0], preferred_element_type=jnp.float32) + b1_ref[0]
    gate = jnp.minimum(v, LIMIT)
    glu = gate * jax.nn.sigmoid(gate * ALPHA)
    up1 = jnp.clip(v, -LIMIT, LIMIT) + 1.0
    h_inter = glu * jnp.roll(up1, -1, axis=1)
    r = jax.lax.broadcasted_iota(jnp.int32, (2 * FF, FF), 0)
    c = jax.lax.broadcasted_iota(jnp.int32, (2 * FF, FF), 1)
    sel = (r == 2 * c).astype(jnp.float32)
    h = jnp.dot(h_inter, sel, preferred_element_type=jnp.float32)
    y = jnp.dot(h, w2_ref[0], preferred_element_type=jnp.float32) + b2_ref[0]
    eidx = jax.lax.broadcasted_iota(jnp.int32, (T, E), 1)
    wcol = jnp.sum(jnp.where(eidx == e, wdense_ref[...], 0.0),
                   axis=1, keepdims=True)
    o_ref[...] += wcol * y


@functools.partial(jax.jit, static_argnames=("interpret",))
def _moe(flat, W_router, b_router, gate_up_proj, gate_up_proj_bias,
         down_proj, down_proj_bias, interpret=False):
    T = flat.shape[0]
    return pl.pallas_call(
        _moe_body,
        grid=(E,),
        in_specs=[
            pl.BlockSpec((T, H), lambda e: (0, 0)),
            pl.BlockSpec((H, E), lambda e: (0, 0)),
            pl.BlockSpec((1, E), lambda e: (0, 0)),
            pl.BlockSpec((1, H, 2 * FF), lambda e: (e, 0, 0)),
            pl.BlockSpec((1, 1, 2 * FF), lambda e: (e, 0, 0)),
            pl.BlockSpec((1, FF, H), lambda e: (e, 0, 0)),
            pl.BlockSpec((1, 1, H), lambda e: (e, 0, 0)),
        ],
        out_specs=pl.BlockSpec((T, H), lambda e: (0, 0)),
        out_shape=jax.ShapeDtypeStruct((T, H), jnp.float32),
        scratch_shapes=[pltpu.VMEM((T, E), jnp.float32)],
        interpret=interpret,
    )(flat, W_router, b_router.reshape(1, E), gate_up_proj,
      gate_up_proj_bias.reshape(E, 1, 2 * FF), down_proj,
      down_proj_bias.reshape(E, 1, H))


def kernel(hidden_states, W_router, b_router, gate_up_proj,
           gate_up_proj_bias, down_proj, down_proj_bias):
    batch = hidden_states.shape[0]
    flat = hidden_states.reshape(-1, H)
    out = _moe(flat, W_router, b_router, gate_up_proj, gate_up_proj_bias,
               down_proj, down_proj_bias)
    return out.reshape(batch, -1, H)
